# baseline (speedup 1.0000x reference)
import jax
import jax.numpy as jnp
from jax.experimental import pallas as pl

HIDDEN_SIZE = 64
TOP_K = 8
EPS = 1e-06
BH_PER = 4


def _obm_kernel(k_ref, v_ref, q_ref, o_ref):
    f32 = jnp.float32
    bf16 = jnp.bfloat16
    G, S, D = k_ref.shape
    I = HIDDEN_SIZE
    Kf = k_ref[...].reshape(G * S, D)
    Vf = v_ref[...].reshape(G * S, D)
    Qf = q_ref[...].reshape(G * S, D)

    iota = jax.lax.broadcasted_iota(jnp.int32, (G * S, D), 1)

    absk = jnp.abs(Kf)
    mk = jnp.max(absk, axis=1, keepdims=True)
    assign = jnp.min(jnp.where(absk == mk, iota, D), axis=1, keepdims=True)
    E2 = (iota == assign).astype(f32)

    absq = jnp.abs(Qf)
    remaining = absq
    m0 = jnp.max(remaining, axis=1, keepdims=True)
    topexp = jnp.zeros((G * S, D), f32)
    expsum = jnp.zeros((G * S, 1), f32)
    for _ in range(TOP_K):
        mt = jnp.max(remaining, axis=1, keepdims=True)
        ft = jnp.min(jnp.where(remaining == mt, iota, D), axis=1, keepdims=True)
        oh = iota == ft
        e = jnp.exp(mt - m0)
        topexp = topexp + jnp.where(oh, e, f32(0))
        expsum = expsum + e
        remaining = jnp.where(oh, -jnp.inf, remaining)

    Kb = Kf.astype(bf16)
    Vb = Vf.astype(bf16)
    Qb = Qf.astype(bf16)

    for j in range(G):
        sl = slice(j * S, (j + 1) * S)
        K_j = Kf[sl]
        Kb_j = Kb[sl]
        Vb_j = Vb[sl]
        Qb_j = Qb[sl]
        Q_j = Qf[sl]
        E2T = E2[sl].T

        W3 = E2T[:, :, None].astype(bf16) * Vb_j[None, :, :]
        M3 = jax.lax.dot_general(W3, Kb_j, (((1,), (0,)), ((), ())),
                                 preferred_element_type=f32)
        num3 = jax.lax.dot_general(M3.astype(bf16), Qb_j,
                                   (((2,), (1,)), ((), ())),
                                   preferred_element_type=f32)

        z3 = E2T[:, :, None] * K_j[None, :, :]
        Z = jnp.sum(z3, axis=1)
        den = jnp.sum(Q_j[:, None, :] * Z[None, :, :], axis=2) + EPS

        P = topexp[sl] / (expsum[sl] * den)
        Pt = P.T
        out_t = jnp.sum(num3 * Pt[:, None, :], axis=0)
        o_ref[j] = out_t.T


@jax.jit
def kernel(keys, values, queries):
    B, H, S, D = keys.shape
    ks = keys.reshape(B * H, S, D)
    vs = values.reshape(B * H, S, D)
    qs = queries.reshape(B * H, S, D)
    spec = pl.BlockSpec((BH_PER, S, D), lambda i: (i, 0, 0))
    out = pl.pallas_call(
        _obm_kernel,
        grid=(B * H // BH_PER,),
        in_specs=[spec, spec, spec],
        out_specs=spec,
        out_shape=jax.ShapeDtypeStruct((B * H, S, D), keys.dtype),
    )(ks, vs, qs)
    return out.reshape(B, H, S, D)


# speedup vs baseline: 1.0060x; 1.0060x over previous
import jax
import jax.numpy as jnp
from jax.experimental import pallas as pl

HIDDEN_SIZE = 64
TOP_K = 8
EPS = 1e-06
BH_PER = 8


def _obm_kernel(k_ref, v_ref, q_ref, o_ref):
    f32 = jnp.float32
    bf16 = jnp.bfloat16
    G, S, D = k_ref.shape
    I = HIDDEN_SIZE
    Kf = k_ref[...].reshape(G * S, D)
    Vf = v_ref[...].reshape(G * S, D)
    Qf = q_ref[...].reshape(G * S, D)

    iota = jax.lax.broadcasted_iota(jnp.int32, (G * S, D), 1)

    absk = jnp.abs(Kf)
    mk = jnp.max(absk, axis=1, keepdims=True)
    assign = jnp.min(jnp.where(absk == mk, iota, D), axis=1, keepdims=True)
    E2 = (iota == assign).astype(f32)

    absq = jnp.abs(Qf)
    remaining = absq
    m0 = jnp.max(remaining, axis=1, keepdims=True)
    topexp = jnp.zeros((G * S, D), f32)
    expsum = jnp.zeros((G * S, 1), f32)
    for _ in range(TOP_K):
        mt = jnp.max(remaining, axis=1, keepdims=True)
        ft = jnp.min(jnp.where(remaining == mt, iota, D), axis=1, keepdims=True)
        oh = iota == ft
        e = jnp.exp(mt - m0)
        topexp = topexp + jnp.where(oh, e, f32(0))
        expsum = expsum + e
        remaining = jnp.where(oh, -jnp.inf, remaining)

    Kb = Kf.astype(bf16)
    Vb = Vf.astype(bf16)
    Qb = Qf.astype(bf16)

    for j in range(G):
        sl = slice(j * S, (j + 1) * S)
        K_j = Kf[sl]
        Kb_j = Kb[sl]
        Vb_j = Vb[sl]
        Qb_j = Qb[sl]
        Q_j = Qf[sl]
        E2T = E2[sl].T

        W3 = E2T[:, :, None].astype(bf16) * Vb_j[None, :, :]
        M3 = jax.lax.dot_general(W3, Kb_j, (((1,), (0,)), ((), ())),
                                 preferred_element_type=f32)
        num3 = jax.lax.dot_general(M3.astype(bf16), Qb_j,
                                   (((2,), (1,)), ((), ())),
                                   preferred_element_type=f32)

        z3 = E2T[:, :, None] * K_j[None, :, :]
        Z = jnp.sum(z3, axis=1)
        den = jnp.sum(Q_j[:, None, :] * Z[None, :, :], axis=2) + EPS

        P = topexp[sl] / (expsum[sl] * den)
        Pt = P.T
        out_t = jnp.sum(num3 * Pt[:, None, :], axis=0)
        o_ref[j] = out_t.T


@jax.jit
def kernel(keys, values, queries):
    B, H, S, D = keys.shape
    ks = keys.reshape(B * H, S, D)
    vs = values.reshape(B * H, S, D)
    qs = queries.reshape(B * H, S, D)
    spec = pl.BlockSpec((BH_PER, S, D), lambda i: (i, 0, 0))
    out = pl.pallas_call(
        _obm_kernel,
        grid=(B * H // BH_PER,),
        in_specs=[spec, spec, spec],
        out_specs=spec,
        out_shape=jax.ShapeDtypeStruct((B * H, S, D), keys.dtype),
    )(ks, vs, qs)
    return out.reshape(B, H, S, D)


# speedup vs baseline: 1.1129x; 1.1063x over previous
import jax
import jax.numpy as jnp
from jax.experimental import pallas as pl

HIDDEN_SIZE = 64
TOP_K = 8
EPS = 1e-06
BH_PER = 8


def _obm_kernel(k_ref, v_ref, q_ref, o_ref):
    f32 = jnp.float32
    bf16 = jnp.bfloat16
    G, S, D = k_ref.shape
    I = HIDDEN_SIZE
    Kf = k_ref[...].reshape(G * S, D)
    Vf = v_ref[...].reshape(G * S, D)
    Qf = q_ref[...].reshape(G * S, D)

    iota = jax.lax.broadcasted_iota(jnp.int32, (G * S, D), 1)

    absk = jnp.abs(Kf)
    mk = jnp.max(absk, axis=1, keepdims=True)
    assign = jnp.min(jnp.where(absk == mk, iota, D), axis=1, keepdims=True)
    E2 = (iota == assign).astype(f32)

    Kb = Kf.astype(bf16)
    Vb = Vf.astype(bf16)
    Qb = Qf.astype(bf16)

    nums = []
    dens = []
    for j in range(G):
        sl = slice(j * S, (j + 1) * S)
        K_j = Kf[sl]
        Kb_j = Kb[sl]
        Vb_j = Vb[sl]
        Qb_j = Qb[sl]
        Q_j = Qf[sl]
        E2T = E2[sl].T

        W3 = E2T[:, :, None].astype(bf16) * Vb_j[None, :, :]
        M3 = jax.lax.dot_general(W3, Kb_j, (((1,), (0,)), ((), ())),
                                 preferred_element_type=f32)
        num3 = jax.lax.dot_general(M3.astype(bf16), Qb_j,
                                   (((2,), (1,)), ((), ())),
                                   preferred_element_type=f32)
        nums.append(num3)

        z3 = E2T[:, :, None] * K_j[None, :, :]
        Z = jnp.sum(z3, axis=1)
        dens.append(jnp.sum(Q_j[:, None, :] * Z[None, :, :], axis=2) + EPS)

    absq = jnp.abs(Qf)
    remaining = absq
    m0 = jnp.max(remaining, axis=1, keepdims=True)
    topexp = jnp.zeros((G * S, D), f32)
    expsum = jnp.zeros((G * S, 1), f32)
    for _ in range(TOP_K):
        mt = jnp.max(remaining, axis=1, keepdims=True)
        ft = jnp.min(jnp.where(remaining == mt, iota, D), axis=1, keepdims=True)
        oh = iota == ft
        e = jnp.exp(mt - m0)
        topexp = topexp + jnp.where(oh, e, f32(0))
        expsum = expsum + e
        remaining = jnp.where(oh, -jnp.inf, remaining)

    for j in range(G):
        sl = slice(j * S, (j + 1) * S)
        P = topexp[sl] / (expsum[sl] * dens[j])
        Pt = P.T
        out_t = jnp.sum(nums[j] * Pt[:, None, :], axis=0)
        o_ref[j] = out_t.T


@jax.jit
def kernel(keys, values, queries):
    B, H, S, D = keys.shape
    ks = keys.reshape(B * H, S, D)
    vs = values.reshape(B * H, S, D)
    qs = queries.reshape(B * H, S, D)
    spec = pl.BlockSpec((BH_PER, S, D), lambda i: (i, 0, 0))
    out = pl.pallas_call(
        _obm_kernel,
        grid=(B * H // BH_PER,),
        in_specs=[spec, spec, spec],
        out_specs=spec,
        out_shape=jax.ShapeDtypeStruct((B * H, S, D), keys.dtype),
    )(ks, vs, qs)
    return out.reshape(B, H, S, D)
